# baseline (speedup 1.0000x reference)
import functools

import jax
import jax.numpy as jnp
from jax import lax
from jax.experimental import pallas as pl
from jax.experimental.pallas import tpu as pltpu
from jax.experimental.pallas import tpu_sc as plsc

NC = 2
NS = 16
NW = NC * NS
CHUNK = 128

RATE_SCALE = 4.0
RATE_LO = 1.0


@functools.lru_cache(maxsize=None)
def _build(B, D):
    b_per_w = B // NW
    n_chunks = b_per_w // CHUNK
    n_groups = b_per_w // 16

    mesh = plsc.VectorSubcoreMesh(core_axis_name="c", subcore_axis_name="s")

    @functools.partial(
        pl.kernel,
        mesh=mesh,
        out_type=jax.ShapeDtypeStruct((B,), jnp.float32),
        compiler_params=pltpu.CompilerParams(
            needs_layout_passes=False, use_tc_tiling_on_sc=False),
        scratch_types=[
            pltpu.VMEM((n_chunks, CHUNK), jnp.int32),
            pltpu.VMEM((n_chunks, CHUNK), jnp.int32),
            pltpu.VMEM((b_per_w, D), jnp.float32),
            pltpu.VMEM((b_per_w, D), jnp.float32),
            pltpu.VMEM((b_per_w,), jnp.float32),
            pltpu.VMEM((b_per_w,), jnp.float32),
            pltpu.VMEM((b_per_w,), jnp.float32),
            pltpu.SemaphoreType.DMA,
        ],
    )
    def mf_kernel(uidx_hbm, iidx_hbm, uemb_hbm, iemb_hbm, ub_hbm, ib_hbm,
                  out_hbm, uidx_v, iidx_v, urows_v, irows_v, ub_v, ib_v,
                  out_v, sem):
        wid = lax.axis_index("c") * NS + lax.axis_index("s")

        pltpu.sync_copy(uidx_hbm.at[pl.ds(wid * n_chunks, n_chunks)], uidx_v)
        pltpu.sync_copy(iidx_hbm.at[pl.ds(wid * n_chunks, n_chunks)], iidx_v)

        copies = []
        for j in range(n_chunks):
            copies.append(pltpu.async_copy(
                uemb_hbm.at[uidx_v.at[j]], urows_v.at[pl.ds(j * CHUNK, CHUNK)], sem))
            copies.append(pltpu.async_copy(
                iemb_hbm.at[iidx_v.at[j]], irows_v.at[pl.ds(j * CHUNK, CHUNK)], sem))
            copies.append(pltpu.async_copy(
                ub_hbm.at[uidx_v.at[j]], ub_v.at[pl.ds(j * CHUNK, CHUNK)], sem))
            copies.append(pltpu.async_copy(
                ib_hbm.at[iidx_v.at[j]], ib_v.at[pl.ds(j * CHUNK, CHUNK)], sem))
        for c in copies:
            c.wait()

        def group_body(g, carry):
            base = g * 16
            rows = lax.iota(jnp.int32, 16) + base
            acc = ub_v[pl.ds(base, 16)] + ib_v[pl.ds(base, 16)]
            for d in range(D):
                dix = jnp.full((16,), d, jnp.int32)
                uc = plsc.load_gather(urows_v, [rows, dix])
                ic = plsc.load_gather(irows_v, [rows, dix])
                acc = acc + uc * ic
            pred = RATE_SCALE / (1.0 + jnp.exp(-acc)) + RATE_LO
            out_v[pl.ds(base, 16)] = pred
            return carry

        lax.fori_loop(0, n_groups, group_body, 0)

        pltpu.sync_copy(out_v, out_hbm.at[pl.ds(wid * b_per_w, b_per_w)])

    return mf_kernel


def kernel(X, user_embeddings, user_bias, item_embeddings, item_bias):
    B = X.shape[0]
    D = user_embeddings.shape[1]
    uidx = X[:, 0].astype(jnp.int32).reshape(B // CHUNK, CHUNK)
    iidx = X[:, 1].astype(jnp.int32).reshape(B // CHUNK, CHUNK)
    ub = user_bias.reshape(-1)
    ib = item_bias.reshape(-1)
    out = _build(B, D)(uidx, iidx, user_embeddings, item_embeddings, ub, ib)
    return out.reshape(B, 1)
